# baseline (speedup 1.0000x reference)
import numpy as np
import jax
import jax.numpy as jnp
from jax import lax
from jax.experimental import pallas as pl
from jax.experimental.pallas import tpu as pltpu

T = 50
D = 50
BS = 4
RI = 12
RC = 16
NA = T * D
NQ = RI + RC
NL = NQ * NA
KT = T + 3

_RV = np.repeat(
    np.where(np.arange(NQ) < RI, np.arange(NQ), np.arange(NQ) - RI),
    NA).astype(np.float32) + 0.5
_INNER = np.repeat((np.arange(NQ) < RI).astype(np.float32), NA)


def _body(x_ref, st_ref, en_ref, rv_ref, in_ref, out_ref, w_ref):
    @pl.when(pl.program_id(0) == 0)
    def _build():
        st = st_ref[...]
        en = en_ref[...]
        rv = rv_ref[...]
        isin = in_ref[...] > 0.5
        ln = jnp.maximum(en - st, 1.0)
        binsz = ln * jnp.where(isin, 1.0 / RI, 1.0 / RC)
        pos = st + binsz * rv
        tv = jnp.where(isin, float(T), 3.0)
        tc = jnp.where(isin, float(T - 1), 2.0)
        valid = (pos >= -1.0) & (pos <= tv)
        pos_c = jnp.minimum(jnp.maximum(pos, 0.0), tc)
        lo = jnp.floor(pos_c)
        hi = jnp.minimum(lo + 1.0, tc)
        w = pos_c - lo
        wlo = jnp.where(valid, 1.0 - w, 0.0)
        whi = jnp.where(valid, w, 0.0)
        base = jnp.where(isin, 0, T)
        lo_i = lo.astype(jnp.int32) + base
        hi_i = hi.astype(jnp.int32) + base
        tio = lax.broadcasted_iota(jnp.int32, (KT, NL), 0)
        w_ref[...] = jnp.where(tio == lo_i, wlo,
                               jnp.where(tio == hi_i, whi, 0.0))

    xb = x_ref[0]

    ip = lax.dot_general(xb, xb, (((0,), (0,)), ((), ())),
                         preferred_element_type=jnp.float32)
    xx = jnp.sum(xb * xb * xb, axis=0, keepdims=True)
    m = jnp.sum(xb, axis=0, keepdims=True) * (1.0 / T)
    score = 2.0 * ip - xx

    jio = lax.broadcasted_iota(jnp.int32, (T, T), 1)
    mb = jnp.broadcast_to(m, (T, T))
    feats = []
    work = score
    for _ in range(3):
        rmax = jnp.max(work, axis=1, keepdims=True)
        cand = jnp.where(work == rmax, jio, T)
        idx = jnp.min(cand, axis=1, keepdims=True)
        onehot = jio == idx
        feats.append(jnp.sum(jnp.where(onehot, mb, 0.0), axis=1, keepdims=True))
        work = jnp.where(onehot, -jnp.inf, work)

    g = jnp.concatenate([xb] + feats, axis=1)
    out_ref[0] = lax.dot_general(g, w_ref[...], (((1,), (0,)), ((), ())),
                                 preferred_element_type=jnp.float32)


def kernel(x, index, anchors):
    del index
    st = jnp.tile(anchors[:NA, 1].reshape(T, D).T.reshape(1, NA), (1, NQ))
    en = jnp.tile(anchors[:NA, 2].reshape(T, D).T.reshape(1, NA), (1, NQ))
    rv = jnp.asarray(_RV).reshape(1, NL)
    isin = jnp.asarray(_INNER).reshape(1, NL)

    out = pl.pallas_call(
        _body,
        grid=(BS,),
        in_specs=[
            pl.BlockSpec((1, T, T), lambda b: (b, 0, 0)),
            pl.BlockSpec((1, NL), lambda b: (0, 0)),
            pl.BlockSpec((1, NL), lambda b: (0, 0)),
            pl.BlockSpec((1, NL), lambda b: (0, 0)),
            pl.BlockSpec((1, NL), lambda b: (0, 0)),
        ],
        out_specs=pl.BlockSpec((1, T, NL), lambda b: (b, 0, 0)),
        out_shape=jax.ShapeDtypeStruct((BS, T, NL), jnp.float32),
        scratch_shapes=[pltpu.VMEM((KT, NL), jnp.float32)],
    )(x, st, en, rv, isin)
    return out.reshape(BS, T * NQ, D, T)


# speedup vs baseline: 1.4696x; 1.4696x over previous
import jax
import jax.numpy as jnp
from jax import lax
from jax.experimental import pallas as pl
from jax.experimental.pallas import tpu as pltpu

T = 50
D = 50
BS = 4
RI = 12
RC = 16
NA = T * D
NQ = RI + RC
KT = T + 3


def _body(x_ref, st_ref, en_ref, out_ref, w_ref, g_ref):
    b = pl.program_id(0)
    q = pl.program_id(1)
    fq = q.astype(jnp.float32)

    @pl.when(b == 0)
    def _build_w():
        st = st_ref[...]
        en = en_ref[...]
        isin = q < RI
        rv = jnp.where(isin, fq, fq - RI) + 0.5
        ln = jnp.maximum(en - st, 1.0)
        binsz = ln * jnp.where(isin, 1.0 / RI, 1.0 / RC)
        pos = st + binsz * rv
        tv = jnp.where(isin, float(T), 3.0)
        tc = jnp.where(isin, float(T - 1), 2.0)
        valid = (pos >= -1.0) & (pos <= tv)
        pos_c = jnp.minimum(jnp.maximum(pos, 0.0), tc)
        lo = jnp.floor(pos_c)
        hi = jnp.minimum(lo + 1.0, tc)
        w = pos_c - lo
        wlo = jnp.where(valid, 1.0 - w, 0.0)
        whi = jnp.where(valid, w, 0.0)
        base = jnp.where(isin, 0, T)
        lo_i = lo.astype(jnp.int32) + base
        hi_i = hi.astype(jnp.int32) + base
        tio = lax.broadcasted_iota(jnp.int32, (KT, NA), 0)
        w_ref[q] = jnp.where(tio == lo_i, wlo,
                             jnp.where(tio == hi_i, whi, 0.0))

    @pl.when(q == 0)
    def _build_g():
        xb = x_ref[0]
        ip = lax.dot_general(xb, xb, (((0,), (0,)), ((), ())),
                             preferred_element_type=jnp.float32)
        xx = jnp.sum(xb * xb * xb, axis=0, keepdims=True)
        m = jnp.sum(xb, axis=0, keepdims=True) * (1.0 / T)
        score = 2.0 * ip - xx

        jio = lax.broadcasted_iota(jnp.int32, (T, T), 1)
        mb = jnp.broadcast_to(m, (T, T))
        feats = []
        work = score
        for _ in range(3):
            rmax = jnp.max(work, axis=1, keepdims=True)
            cand = jnp.where(work == rmax, jio, T)
            idx = jnp.min(cand, axis=1, keepdims=True)
            onehot = jio == idx
            feats.append(jnp.sum(jnp.where(onehot, mb, 0.0),
                                 axis=1, keepdims=True))
            work = jnp.where(onehot, -jnp.inf, work)
        g_ref[...] = jnp.concatenate([xb] + feats, axis=1)

    res = lax.dot_general(g_ref[...], w_ref[q], (((1,), (0,)), ((), ())),
                          preferred_element_type=jnp.float32)
    out_ref[0, :, 0] = res.reshape(T, D, T)


def kernel(x, index, anchors):
    del index
    st = anchors[:NA, 1].reshape(T, D).T.reshape(1, NA)
    en = anchors[:NA, 2].reshape(T, D).T.reshape(1, NA)

    out = pl.pallas_call(
        _body,
        grid=(BS, NQ),
        in_specs=[
            pl.BlockSpec((1, T, T), lambda b, q: (b, 0, 0)),
            pl.BlockSpec((1, NA), lambda b, q: (0, 0)),
            pl.BlockSpec((1, NA), lambda b, q: (0, 0)),
        ],
        out_specs=pl.BlockSpec((1, T, 1, D, T), lambda b, q: (b, 0, q, 0, 0)),
        out_shape=jax.ShapeDtypeStruct((BS, T, NQ, D, T), jnp.float32),
        scratch_shapes=[
            pltpu.VMEM((NQ, KT, NA), jnp.float32),
            pltpu.VMEM((T, KT), jnp.float32),
        ],
    )(x, st, en)
    return out.reshape(BS, T * NQ, D, T)


# speedup vs baseline: 1.7484x; 1.1897x over previous
import jax
import jax.numpy as jnp
from jax import lax
from jax.experimental import pallas as pl
from jax.experimental.pallas import tpu as pltpu

T = 50
D = 50
BS = 4
RI = 12
RC = 16
NA = T * D
NQ = RI + RC
KT = T + 3
QG = 7
NG = NQ // QG


def _body(x_ref, st_ref, en_ref, out_ref, w_ref, g_ref):
    b = pl.program_id(0)
    qs = pl.program_id(1)

    @pl.when(b == 0)
    def _build_w():
        st = st_ref[...]
        en = en_ref[...]
        ln = jnp.maximum(en - st, 1.0)
        tio = lax.broadcasted_iota(jnp.int32, (KT, NA), 0)
        for r in range(QG):
            q = qs * QG + r
            fq = q.astype(jnp.float32)
            isin = q < RI
            rv = jnp.where(isin, fq, fq - RI) + 0.5
            binsz = ln * jnp.where(isin, 1.0 / RI, 1.0 / RC)
            pos = st + binsz * rv
            tv = jnp.where(isin, float(T), 3.0)
            tc = jnp.where(isin, float(T - 1), 2.0)
            valid = (pos >= -1.0) & (pos <= tv)
            pos_c = jnp.minimum(jnp.maximum(pos, 0.0), tc)
            lo = jnp.floor(pos_c)
            hi = jnp.minimum(lo + 1.0, tc)
            w = pos_c - lo
            wlo = jnp.where(valid, 1.0 - w, 0.0)
            whi = jnp.where(valid, w, 0.0)
            base = jnp.where(isin, 0, T)
            lo_i = lo.astype(jnp.int32) + base
            hi_i = hi.astype(jnp.int32) + base
            w_ref[q] = jnp.where(tio == lo_i, wlo,
                                 jnp.where(tio == hi_i, whi, 0.0))

    @pl.when(qs == 0)
    def _build_g():
        xb = x_ref[0]
        ip = lax.dot_general(xb, xb, (((0,), (0,)), ((), ())),
                             preferred_element_type=jnp.float32)
        xx = jnp.sum(xb * xb * xb, axis=0, keepdims=True)
        m = jnp.sum(xb, axis=0, keepdims=True) * (1.0 / T)
        score = 2.0 * ip - xx

        jio = lax.broadcasted_iota(jnp.int32, (T, T), 1)
        mb = jnp.broadcast_to(m, (T, T))
        feats = []
        work = score
        for _ in range(3):
            rmax = jnp.max(work, axis=1, keepdims=True)
            cand = jnp.where(work == rmax, jio, T)
            idx = jnp.min(cand, axis=1, keepdims=True)
            onehot = jio == idx
            feats.append(jnp.sum(jnp.where(onehot, mb, 0.0),
                                 axis=1, keepdims=True))
            work = jnp.where(onehot, -jnp.inf, work)
        g_ref[...] = jnp.concatenate([xb] + feats, axis=1)

    g = g_ref[...]
    for r in range(QG):
        res = lax.dot_general(g, w_ref[qs * QG + r],
                              (((1,), (0,)), ((), ())),
                              preferred_element_type=jnp.float32)
        out_ref[0, :, r] = res.reshape(T, D, T)


def kernel(x, index, anchors):
    del index
    st = anchors[:NA, 1].reshape(T, D).T.reshape(1, NA)
    en = anchors[:NA, 2].reshape(T, D).T.reshape(1, NA)

    out = pl.pallas_call(
        _body,
        grid=(BS, NG),
        in_specs=[
            pl.BlockSpec((1, T, T), lambda b, qs: (b, 0, 0)),
            pl.BlockSpec((1, NA), lambda b, qs: (0, 0)),
            pl.BlockSpec((1, NA), lambda b, qs: (0, 0)),
        ],
        out_specs=pl.BlockSpec((1, T, QG, D, T),
                               lambda b, qs: (b, 0, qs, 0, 0)),
        out_shape=jax.ShapeDtypeStruct((BS, T, NQ, D, T), jnp.float32),
        scratch_shapes=[
            pltpu.VMEM((NQ, KT, NA), jnp.float32),
            pltpu.VMEM((T, KT), jnp.float32),
        ],
    )(x, st, en)
    return out.reshape(BS, T * NQ, D, T)
